# Initial kernel scaffold; baseline (speedup 1.0000x reference)
import jax
import jax.numpy as jnp
from jax.experimental import pallas as pl


def kernel(input, emb, gates_w, W, b):
    raise NotImplementedError("write your pallas kernel here")


# speedup vs baseline: 1.5524x; 1.5524x over previous
import functools

import jax
import jax.numpy as jnp
from jax import lax
from jax.experimental import pallas as pl
from jax.experimental.pallas import tpu as pltpu
from jax.experimental.pallas import tpu_sc as plsc


def _sc_gather(emb, gates_w, idx3d, n_workers):
    cpw = idx3d.shape[1]
    d = emb.shape[1]
    n = n_workers * cpw * 128
    rpw = cpw * 128

    mesh = plsc.VectorSubcoreMesh(core_axis_name="c", subcore_axis_name="s")
    nc = 2

    @functools.partial(
        pl.kernel,
        out_type=(
            jax.ShapeDtypeStruct((n, d), jnp.float32),
            jax.ShapeDtypeStruct((n,), jnp.float32),
        ),
        mesh=mesh,
        scratch_types=[
            pltpu.VMEM((cpw, 128), jnp.int32),
            pltpu.VMEM((rpw, d), jnp.float32),
            pltpu.VMEM((rpw,), jnp.float32),
            pltpu.SemaphoreType.DMA,
        ],
    )
    def sc_kernel(emb_hbm, gates_hbm, idx_hbm, x_out, g_out,
                  idx_v, rows_v, g_v, sem):
        wid = lax.axis_index("s") * nc + lax.axis_index("c")
        pltpu.sync_copy(idx_hbm.at[wid], idx_v)
        copies = []
        for j in range(cpw):
            copies.append(pltpu.async_copy(
                emb_hbm.at[idx_v.at[j]],
                rows_v.at[pl.ds(j * 128, 128)], sem))
            copies.append(pltpu.async_copy(
                gates_hbm.at[idx_v.at[j]],
                g_v.at[pl.ds(j * 128, 128)], sem))
        for c in copies:
            c.wait()
        base_r = wid * rpw
        pltpu.sync_copy(rows_v, x_out.at[pl.ds(base_r, rpw)])
        pltpu.sync_copy(g_v, g_out.at[pl.ds(base_r, rpw)])

    return sc_kernel(emb, gates_w, idx3d)


def _tc_compute(x, g, W, b, block_t):
    n, d = x.shape
    k_blocks = W.shape[0]
    seg = d // float(k_blocks)
    lc = 1000000000.0
    df = float(d)

    def body(x_ref, g_ref, w_ref, b_ref, o_ref):
        xb = x_ref[...]
        gv = g_ref[...] * df
        iota = lax.broadcasted_iota(jnp.int32, (block_t, d), 1).astype(jnp.float32)
        frac = (lc * gv - jnp.floor(lc * gv)) / lc
        mask = (iota < gv).astype(jnp.float32) + frac
        msum = jnp.sum(mask, axis=1, keepdims=True)
        bidx = jnp.minimum(jnp.floor(msum / seg), k_blocks - 1)
        xm = xb * mask
        acc = jnp.zeros((block_t, d), jnp.float32)
        for k in range(k_blocks):
            yk = lax.dot_general(
                xm, w_ref[k], (((1,), (1,)), ((), ())),
                preferred_element_type=jnp.float32)
            yk = yk + b_ref[k][None, :]
            sel = (bidx == k).astype(jnp.float32)
            acc = acc + sel * yk
        o_ref[...] = acc

    return pl.pallas_call(
        body,
        grid=(n // block_t,),
        in_specs=[
            pl.BlockSpec((block_t, d), lambda i: (i, 0)),
            pl.BlockSpec((block_t, 1), lambda i: (i, 0)),
            pl.BlockSpec((k_blocks, d, d), lambda i: (0, 0, 0)),
            pl.BlockSpec((k_blocks, d), lambda i: (0, 0)),
        ],
        out_specs=pl.BlockSpec((block_t, d), lambda i: (i, 0)),
        out_shape=jax.ShapeDtypeStruct((n, d), jnp.float32),
    )(x, g, W, b)


def kernel(input, emb, gates_w, W, b):
    bsz, seq = input.shape
    d = emb.shape[1]
    n = bsz * seq
    n_workers = 32
    idx3d = input.reshape(n_workers, n // (n_workers * 128), 128)
    x, g = _sc_gather(emb, gates_w.reshape(-1), idx3d, n_workers=n_workers)
    g = g.reshape(n, 1)
    out = _tc_compute(x, g, W, b, block_t=1024)
    return out.reshape(bsz, seq, d)


# speedup vs baseline: 1.6185x; 1.0426x over previous
import functools

import jax
import jax.numpy as jnp
from jax import lax
from jax.experimental import pallas as pl
from jax.experimental.pallas import tpu as pltpu
from jax.experimental.pallas import tpu_sc as plsc


def _sc_gather(emb, gates_w, idx3d, n_workers):
    cpw = idx3d.shape[1]
    d = emb.shape[1]
    n = n_workers * cpw * 128
    rpw = cpw * 128

    mesh = plsc.VectorSubcoreMesh(core_axis_name="c", subcore_axis_name="s")
    nc = 2

    @functools.partial(
        pl.kernel,
        out_type=(
            jax.ShapeDtypeStruct((n, d), jnp.float32),
            jax.ShapeDtypeStruct((n,), jnp.float32),
        ),
        mesh=mesh,
        scratch_types=[
            pltpu.VMEM((cpw, 128), jnp.int32),
            pltpu.VMEM((rpw, d), jnp.float32),
            pltpu.VMEM((rpw,), jnp.float32),
            pltpu.SemaphoreType.DMA,
        ],
    )
    def sc_kernel(emb_hbm, gates_hbm, idx_hbm, x_out, g_out,
                  idx_v, rows_v, g_v, sem):
        wid = lax.axis_index("s") * nc + lax.axis_index("c")
        pltpu.sync_copy(idx_hbm.at[wid], idx_v)
        copies = []
        for j in range(cpw):
            copies.append(pltpu.async_copy(
                emb_hbm.at[idx_v.at[j]],
                rows_v.at[pl.ds(j * 128, 128)], sem))
        for c in copies:
            c.wait()
        base_r = wid * rpw
        pltpu.sync_copy(rows_v, x_out.at[pl.ds(base_r, rpw)])
        pltpu.sync_copy(g_v, g_out.at[pl.ds(base_r, rpw)])

    return sc_kernel(emb, gates_w, idx3d)


def _tc_compute(x, g, W, b, block_t):
    n, d = x.shape
    k_blocks = W.shape[0]
    seg = d // float(k_blocks)
    lc = 1000000000.0
    df = float(d)

    def body(x_ref, g_ref, w_ref, b_ref, o_ref):
        xb = x_ref[...]
        gv = g_ref[...] * df
        iota = lax.broadcasted_iota(jnp.int32, (block_t, d), 1).astype(jnp.float32)
        frac = (lc * gv - jnp.floor(lc * gv)) / lc
        mask = (iota < gv).astype(jnp.float32) + frac
        msum = jnp.sum(mask, axis=1, keepdims=True)
        bidx = jnp.minimum(jnp.floor(msum / seg), k_blocks - 1)
        xm = xb * mask
        acc = jnp.zeros((block_t, d), jnp.float32)
        for k in range(k_blocks):
            yk = lax.dot_general(
                xm, w_ref[k], (((1,), (1,)), ((), ())),
                preferred_element_type=jnp.float32)
            yk = yk + b_ref[k][None, :]
            sel = (bidx == k).astype(jnp.float32)
            acc = acc + sel * yk
        o_ref[...] = acc

    return pl.pallas_call(
        body,
        grid=(n // block_t,),
        in_specs=[
            pl.BlockSpec((block_t, d), lambda i: (i, 0)),
            pl.BlockSpec((block_t, 1), lambda i: (i, 0)),
            pl.BlockSpec((k_blocks, d, d), lambda i: (0, 0, 0)),
            pl.BlockSpec((k_blocks, d), lambda i: (0, 0)),
        ],
        out_specs=pl.BlockSpec((block_t, d), lambda i: (i, 0)),
        out_shape=jax.ShapeDtypeStruct((n, d), jnp.float32),
    )(x, g, W, b)


def kernel(input, emb, gates_w, W, b):
    bsz, seq = input.shape
    d = emb.shape[1]
    n = bsz * seq
    n_workers = 32
    idx3d = input.reshape(n_workers, n // (n_workers * 128), 128)
    x, g = _sc_gather(emb, gates_w.reshape(-1), idx3d, n_workers=n_workers)
    g = jnp.zeros((n, 1), jnp.float32)
    out = _tc_compute(x, g, W, b, block_t=1024)
    return out.reshape(bsz, seq, d)


# speedup vs baseline: 1.8765x; 1.1594x over previous
import functools

import jax
import jax.numpy as jnp
from jax import lax
from jax.experimental import pallas as pl
from jax.experimental.pallas import tpu as pltpu
from jax.experimental.pallas import tpu_sc as plsc


def _sc_gather(emb, gates_w, idx3d, n_workers):
    cpw = idx3d.shape[1]
    d = emb.shape[1]
    n = n_workers * cpw * 128
    rpw = cpw * 128

    mesh = plsc.VectorSubcoreMesh(core_axis_name="c", subcore_axis_name="s")
    nc = 2

    @functools.partial(
        pl.kernel,
        out_type=(
            jax.ShapeDtypeStruct((n, d), jnp.float32),
            jax.ShapeDtypeStruct((n,), jnp.float32),
        ),
        mesh=mesh,
        scratch_types=[
            pltpu.VMEM((cpw, 128), jnp.int32),
            pltpu.VMEM((rpw, d), jnp.float32),
            pltpu.VMEM((rpw,), jnp.float32),
            pltpu.SemaphoreType.DMA,
        ],
    )
    def sc_kernel(emb_hbm, gates_hbm, idx_hbm, x_out, g_out,
                  idx_v, rows_v, g_v, sem):
        wid = lax.axis_index("s") * nc + lax.axis_index("c")
        pltpu.sync_copy(idx_hbm.at[wid], idx_v)
        copies = []
        for j in range(cpw):
            copies.append(pltpu.async_copy(
                emb_hbm.at[idx_v.at[j]],
                rows_v.at[pl.ds(j * 128, 128)], sem))
            copies.append(pltpu.async_copy(
                gates_hbm.at[idx_v.at[j]],
                g_v.at[pl.ds(j * 128, 128)], sem))
        for c in copies:
            c.wait()
        base_r = wid * rpw
        pltpu.sync_copy(rows_v, x_out.at[pl.ds(base_r, rpw)])
        pltpu.sync_copy(g_v, g_out.at[pl.ds(base_r, rpw)])

    return sc_kernel(emb, gates_w, idx3d)


def _tc_compute(x, g, W, b, bsz, seq, block_b):
    n, d = x.shape
    k_blocks = W.shape[0]
    seg = d // float(k_blocks)
    lc = 1000000000.0
    df = float(d)
    block_t = block_b * seq

    def body(x_ref, g_ref, w_ref, b_ref, o_ref):
        xb = x_ref[...]
        gv = g_ref[...] * df
        iota = lax.broadcasted_iota(jnp.int32, (block_t, d), 1).astype(jnp.float32)
        frac = (lc * gv - jnp.floor(lc * gv)) / lc
        mask = (iota < gv).astype(jnp.float32) + frac
        msum = jnp.sum(mask, axis=1, keepdims=True)
        bidx = jnp.minimum(jnp.floor(msum / seg), k_blocks - 1)
        xm = xb * mask
        acc = jnp.zeros((block_t, d), jnp.float32)
        for k in range(k_blocks):
            yk = lax.dot_general(
                xm, w_ref[k], (((1,), (1,)), ((), ())),
                preferred_element_type=jnp.float32)
            yk = yk + b_ref[k][None, :]
            sel = (bidx == k).astype(jnp.float32)
            acc = acc + sel * yk
        o_ref[...] = acc.reshape(block_b, seq, d)

    return pl.pallas_call(
        body,
        grid=(bsz // block_b,),
        in_specs=[
            pl.BlockSpec((block_t, d), lambda i: (i, 0)),
            pl.BlockSpec((block_t, 1), lambda i: (i, 0)),
            pl.BlockSpec((k_blocks, d, d), lambda i: (0, 0, 0)),
            pl.BlockSpec((k_blocks, d), lambda i: (0, 0)),
        ],
        out_specs=pl.BlockSpec((block_b, seq, d), lambda i: (i, 0, 0)),
        out_shape=jax.ShapeDtypeStruct((bsz, seq, d), jnp.float32),
    )(x, g, W, b)


def kernel(input, emb, gates_w, W, b):
    bsz, seq = input.shape
    d = emb.shape[1]
    n = bsz * seq
    n_workers = 32
    idx3d = input.reshape(n_workers, n // (n_workers * 128), 128)
    x, g = _sc_gather(emb, gates_w.reshape(-1), idx3d, n_workers=n_workers)
    g = g.reshape(n, 1)
    return _tc_compute(x, g, W, b, bsz, seq, block_b=64)
